# baseline (speedup 1.0000x reference)
import functools
import math

import jax
import jax.numpy as jnp
from jax import lax
from jax.experimental import pallas as pl
from jax.experimental.pallas import tpu as pltpu
from jax.experimental.pallas import tpu_sc as plsc

D_MODEL = 64
SCALE = math.sqrt(D_MODEL)

NC = 2
NS = 16
NW = NC * NS

SB = 128
NBUF = 4
LANES = 16


def _emb_body(x4_hbm, table_hbm, out_hbm, idx_v, pbuf, gbufs, tbufs,
              gsems, ssems):
    wid = lax.axis_index("s") * NC + lax.axis_index("c")
    ntok = x4_hbm.shape[0] * x4_hbm.shape[2]

    pltpu.sync_copy(x4_hbm.at[:, wid], idx_v)

    def start_gather(b, t):
        rb = t // 8
        rr = t % 8
        pltpu.async_copy(table_hbm.at[idx_v.at[rb, rr]], gbufs[b], gsems[b])

    def start_store(b, t):
        pltpu.async_copy(tbufs[b], out_hbm.at[t, :, wid], ssems[b])

    def wait_store(b, t):
        pltpu.make_async_copy(tbufs[b], out_hbm.at[t, :, wid],
                              ssems[b]).wait()

    for b in range(NBUF - 1):
        start_gather(b, b)

    rows16 = [lax.iota(jnp.int32, LANES) + (icc * LANES)
              for icc in range(SB // LANES)]

    def round_body(r):
        for b in range(NBUF):
            t = r * NBUF + b
            rb = t // 8
            rr = t % 8

            pltpu.make_async_copy(table_hbm.at[idx_v.at[rb, rr]], gbufs[b],
                                  gsems[b]).wait()

            def repack_row(row, _):
                for c in range(D_MODEL // LANES):
                    sl = pl.ds(c * LANES, LANES)
                    pbuf[row, sl] = gbufs[b][row, sl] * SCALE
                return 0

            lax.fori_loop(0, SB, repack_row, 0, unroll=4)

            def feat_row(f, _):
                fb = f // 8
                fr = f % 8
                fcol = jnp.broadcast_to(f, (LANES,)).astype(jnp.int32)
                for icc in range(SB // LANES):
                    val = plsc.load_gather(pbuf, [rows16[icc], fcol])
                    tbufs[b][fb, fr, pl.ds(icc * LANES, LANES)] = val
                return 0

            lax.fori_loop(0, D_MODEL, feat_row, 0, unroll=2)

            start_store(b, t)

            bp = (b - 1) % NBUF
            tp = t - 1

            @pl.when(tp >= 0)
            def _():
                wait_store(bp, tp)

            @pl.when(tp + NBUF < ntok)
            def _():
                start_gather(bp, tp + NBUF)

    pl.loop(0, ntok // NBUF)(round_body)

    wait_store((ntok - 1) % NBUF, ntok - 1)


def _tr_body(tabt_hbm, out_hbm, tbuf, tbuf2, obuf):
    wid = lax.axis_index("s") * NC + lax.axis_index("c")
    nv = tabt_hbm.shape[1]
    nblk = nv // SB
    lo = wid * nblk // NW
    hi = (wid + 1) * nblk // NW

    iot = lax.iota(jnp.int32, LANES)
    fvecs = [(c % 4) * LANES + iot for c in range(8)]

    def do_block(ref, nu):
        def pair_row(u, _):
            v0 = jnp.broadcast_to(2 * u, (LANES,)).astype(jnp.int32)
            v1 = v0 + 1
            for c in range(8):
                vv = v0 if c < 4 else v1
                val = plsc.load_gather(ref, [fvecs[c], vv])
                obuf[u, pl.ds((c % 4) * LANES + (0 if c < 4 else D_MODEL),
                              LANES)] = val
            return 0

        lax.fori_loop(0, nu, pair_row, 0, unroll=2)

    def block_loop(b):
        pltpu.sync_copy(tabt_hbm.at[:, pl.ds(b * SB, SB)],
                        tbuf.at[:, pl.ds(0, SB)])
        do_block(tbuf, D_MODEL)
        pltpu.sync_copy(obuf, out_hbm.at[pl.ds(b * D_MODEL, D_MODEL)])

    pl.loop(lo, hi)(block_loop)

    rem = nv - nblk * SB
    if rem > 0:
        @pl.when(wid == NW - 1)
        def _():
            pltpu.sync_copy(tabt_hbm.at[:, pl.ds(nblk * SB, rem)], tbuf2)
            do_block(tbuf2, rem // 2)
            pltpu.sync_copy(obuf.at[pl.ds(0, rem // 2)],
                            out_hbm.at[pl.ds(nblk * D_MODEL, rem // 2)])


@jax.jit
def _tr_call(tabt):
    nv = tabt.shape[1]
    mesh = plsc.VectorSubcoreMesh(core_axis_name="c", subcore_axis_name="s",
                                  num_cores=NC, num_subcores=NS)
    kern = pl.kernel(
        _tr_body,
        out_type=jax.ShapeDtypeStruct((nv // 2, 2 * D_MODEL), jnp.float32),
        mesh=mesh,
        scratch_types=[
            pltpu.VMEM((D_MODEL, SB + 1), jnp.float32),
            pltpu.VMEM((D_MODEL, nv % SB if nv % SB else LANES), jnp.float32),
            pltpu.VMEM((D_MODEL, 2 * D_MODEL), jnp.float32),
        ],
        compiler_params=pltpu.CompilerParams(needs_layout_passes=False),
    )
    return kern(tabt)


@jax.jit
def _emb_call(x4, table):
    ntok = x4.shape[0] * x4.shape[2]
    nsb = x4.shape[1]
    mesh = plsc.VectorSubcoreMesh(core_axis_name="c", subcore_axis_name="s",
                                  num_cores=NC, num_subcores=NS)
    scratch = (
        [pltpu.VMEM((x4.shape[0], x4.shape[2], SB), jnp.int32)]
        + [pltpu.VMEM((SB, D_MODEL + 1), jnp.float32)]
        + [[pltpu.VMEM((SB, D_MODEL), jnp.float32) for _ in range(NBUF)]]
        + [[pltpu.VMEM((D_MODEL // 8, 8, SB), jnp.float32)
            for _ in range(NBUF)]]
        + [[pltpu.SemaphoreType.DMA for _ in range(NBUF)]]
        + [[pltpu.SemaphoreType.DMA for _ in range(NBUF)]]
    )
    kern = pl.kernel(
        _emb_body,
        out_type=jax.ShapeDtypeStruct(
            (ntok, D_MODEL // 8, nsb, 8, SB), jnp.float32),
        mesh=mesh,
        scratch_types=scratch,
        compiler_params=pltpu.CompilerParams(use_tc_tiling_on_sc=False,
                                             needs_layout_passes=False),
    )
    return kern(x4, table)


def kernel(x, table):
    nseq, ntok = x.shape
    x4 = x.T.reshape(ntok // 8, 8, nseq // SB, SB).transpose(0, 2, 1, 3)
    tab_rm = _tr_call(table.T).reshape(-1, D_MODEL)
    out5 = _emb_call(x4, tab_rm)
    return out5.transpose(2, 4, 0, 1, 3).reshape(nseq, ntok, D_MODEL)


# speedup vs baseline: 2.3184x; 2.3184x over previous
import functools
import math

import jax
import jax.numpy as jnp
from jax import lax
from jax.experimental import pallas as pl
from jax.experimental.pallas import tpu as pltpu
from jax.experimental.pallas import tpu_sc as plsc

D_MODEL = 64
SCALE = math.sqrt(D_MODEL)

NC = 2
NS = 16
NW = NC * NS

SB = 128
NBUF = 4
LANES = 16


def _emb_body(x4_hbm, table_hbm, out_hbm, idx_v, bufs, gsems, ssems):
    wid = lax.axis_index("s") * NC + lax.axis_index("c")
    ntok = x4_hbm.shape[0] * x4_hbm.shape[2]
    seq0 = wid * SB

    pltpu.sync_copy(x4_hbm.at[:, wid], idx_v)

    def start_gather(b, t):
        rb = t // 8
        rr = t % 8
        pltpu.async_copy(table_hbm.at[idx_v.at[rb, rr]], bufs[b], gsems[b])

    def start_store(b, t):
        pltpu.async_copy(bufs[b], out_hbm.at[pl.ds(seq0, SB), t], ssems[b])

    def wait_store(b, t):
        pltpu.make_async_copy(bufs[b], out_hbm.at[pl.ds(seq0, SB), t],
                              ssems[b]).wait()

    for b in range(NBUF - 1):
        start_gather(b, b)

    def round_body(r):
        for b in range(NBUF):
            t = r * NBUF + b

            rb = t // 8
            rr = t % 8
            pltpu.make_async_copy(table_hbm.at[idx_v.at[rb, rr]], bufs[b],
                                  gsems[b]).wait()

            def scale_row(row, _):
                for c in range(D_MODEL // LANES):
                    sl = pl.ds(c * LANES, LANES)
                    bufs[b][row, sl] = bufs[b][row, sl] * SCALE
                return 0

            lax.fori_loop(0, SB, scale_row, 0, unroll=4)

            start_store(b, t)

            bp = (b - 1) % NBUF
            tp = t - 1

            @pl.when(tp >= 0)
            def _():
                wait_store(bp, tp)

            @pl.when(tp + NBUF < ntok)
            def _():
                start_gather(bp, tp + NBUF)

    pl.loop(0, ntok // NBUF)(round_body)

    wait_store((ntok - 1) % NBUF, ntok - 1)


@jax.jit
def _emb_call(x4, table):
    ntok = x4.shape[0] * x4.shape[2]
    nseq = x4.shape[1] * x4.shape[3]
    mesh = plsc.VectorSubcoreMesh(core_axis_name="c", subcore_axis_name="s",
                                  num_cores=NC, num_subcores=NS)
    scratch = (
        [pltpu.VMEM((x4.shape[0], x4.shape[2], SB), jnp.int32)]
        + [[pltpu.VMEM((SB, D_MODEL), jnp.float32) for _ in range(NBUF)]]
        + [[pltpu.SemaphoreType.DMA for _ in range(NBUF)]]
        + [[pltpu.SemaphoreType.DMA for _ in range(NBUF)]]
    )
    kern = pl.kernel(
        _emb_body,
        out_type=jax.ShapeDtypeStruct((nseq, ntok, D_MODEL), jnp.float32),
        mesh=mesh,
        scratch_types=scratch,
        compiler_params=pltpu.CompilerParams(use_tc_tiling_on_sc=False),
    )
    return kern(x4, table)


def kernel(x, table):
    nseq, ntok = x.shape
    x4 = x.T.reshape(ntok // 8, 8, nseq // SB, SB).transpose(0, 2, 1, 3)
    return _emb_call(x4, table)
